# baseline (speedup 1.0000x reference)
import functools

import jax
import jax.numpy as jnp
from jax import lax
from jax.experimental import pallas as pl
from jax.experimental.pallas import tpu as pltpu
from jax.experimental.pallas import tpu_sc as plsc

_NC = 2
_NS = 16
_NW = _NC * _NS
_NBUF = 4


def _sc_concat_embed(table, idx, feature):
    B, T, D = feature.shape
    b_per_w = B // _NW
    mesh = plsc.VectorSubcoreMesh(
        core_axis_name="c", subcore_axis_name="s",
        num_cores=_NC, num_subcores=_NS,
    )

    @functools.partial(
        pl.kernel,
        out_type=jax.ShapeDtypeStruct((B, T + 1, D), feature.dtype),
        mesh=mesh,
        compiler_params=pltpu.CompilerParams(use_tc_tiling_on_sc=False),
        scratch_types=[
            pltpu.VMEM((b_per_w,), jnp.int32),
            pltpu.VMEM((b_per_w, D), jnp.float32),
            pltpu.VMEM((_NBUF, T + 1, D), jnp.float32),
            pltpu.SemaphoreType.DMA,
            pltpu.SemaphoreType.DMA,
            pltpu.SemaphoreType.DMA,
        ],
    )
    def body(table_hbm, idx_hbm, feat_hbm, out_hbm,
             idx_v, emb_v, buf, gsem, s_in, s_out):
        wid = lax.axis_index("s") * _NC + lax.axis_index("c")
        base = wid * b_per_w
        pltpu.sync_copy(idx_hbm.at[pl.ds(base, b_per_w)], idx_v)
        pltpu.async_copy(table_hbm.at[idx_v], emb_v, gsem).wait()

        in_copies = [None] * b_per_w
        out_copies = [None] * b_per_w

        def issue_in(i):
            slot = i % _NBUF
            bb = base + i
            in_copies[i] = pltpu.async_copy(
                feat_hbm.at[bb], buf.at[slot, pl.ds(1, T)], s_in)
            for j in range(D // 16):
                buf[slot, 0, pl.ds(j * 16, 16)] = emb_v[i, pl.ds(j * 16, 16)]

        for i in range(_NBUF):
            issue_in(i)
        for i in range(b_per_w):
            in_copies[i].wait()
            slot = i % _NBUF
            out_copies[i] = pltpu.async_copy(
                buf.at[slot], out_hbm.at[base + i], s_out)
            nxt = i + _NBUF
            if nxt < b_per_w:
                out_copies[nxt - _NBUF].wait()
                issue_in(nxt)
        for i in range(b_per_w - _NBUF, b_per_w):
            out_copies[i].wait()

    return body(table, idx, feature)


def kernel(feature, index_value, embedding_table):
    idx = index_value.astype(jnp.int32)
    return _sc_concat_embed(embedding_table, idx, feature)


# speedup vs baseline: 1.6251x; 1.6251x over previous
import functools

import jax
import jax.numpy as jnp
from jax import lax
from jax.experimental import pallas as pl
from jax.experimental.pallas import tpu as pltpu
from jax.experimental.pallas import tpu_sc as plsc

_NC = 2
_NS = 16
_NW = _NC * _NS


def _sc_gather(table, idx):
    B, = idx.shape
    V, D = table.shape
    b_per_w = B // _NW
    mesh = plsc.VectorSubcoreMesh(
        core_axis_name="c", subcore_axis_name="s",
        num_cores=_NC, num_subcores=_NS,
    )

    @functools.partial(
        pl.kernel,
        out_type=jax.ShapeDtypeStruct((B, D), table.dtype),
        mesh=mesh,
        scratch_types=[
            pltpu.VMEM((b_per_w,), jnp.int32),
            pltpu.VMEM((b_per_w, D), jnp.float32),
            pltpu.SemaphoreType.DMA,
        ],
    )
    def gather_kernel(table_hbm, idx_hbm, out_hbm, idx_v, rows_v, sem):
        wid = lax.axis_index("s") * _NC + lax.axis_index("c")
        base = wid * b_per_w
        pltpu.sync_copy(idx_hbm.at[pl.ds(base, b_per_w)], idx_v)
        pltpu.async_copy(table_hbm.at[idx_v], rows_v, sem).wait()
        pltpu.sync_copy(rows_v, out_hbm.at[pl.ds(base, b_per_w)])

    return gather_kernel(table, idx)


def _concat_body(emb_ref, feat_ref, out_ref):
    out_ref[:, 0:1, :] = emb_ref[...]
    out_ref[:, 1:, :] = feat_ref[...]


def _tc_concat(emb, feature, block_b=128):
    B, T, D = feature.shape
    emb3 = emb.reshape(B, 1, D)
    return pl.pallas_call(
        _concat_body,
        grid=(B // block_b,),
        in_specs=[
            pl.BlockSpec((block_b, 1, D), lambda b: (b, 0, 0)),
            pl.BlockSpec((block_b, T, D), lambda b: (b, 0, 0)),
        ],
        out_specs=pl.BlockSpec((block_b, T + 1, D), lambda b: (b, 0, 0)),
        out_shape=jax.ShapeDtypeStruct((B, T + 1, D), feature.dtype),
    )(emb3, feature)


def kernel(feature, index_value, embedding_table):
    idx = index_value.astype(jnp.int32)
    emb = _sc_gather(embedding_table, idx)
    return _tc_concat(emb, feature)
